# baseline (speedup 1.0000x reference)
import functools

import jax
import jax.numpy as jnp
from jax import lax
from jax.experimental import pallas as pl
from jax.experimental.pallas import tpu as pltpu
from jax.experimental.pallas import tpu_sc as plsc

_NC, _NS = 2, 16
_NW = _NC * _NS
_G = 32
_KG = 8
_NR = 4
_L = 16


def kernel(input_ids, word_table, pos_table):
    B, S = input_ids.shape
    V, D = word_table.shape
    R = _KG * _G
    assert _NW * R == S

    pos_r = pos_table.reshape(S // _G, _G, D)

    mesh = plsc.VectorSubcoreMesh(core_axis_name="c", subcore_axis_name="s")

    scratch = (
        [pltpu.VMEM((R,), jnp.int32) for _ in range(B)]
        + [pltpu.VMEM((_G, D), jnp.float32)
           for _ in range(_NR * B)]
        + [pltpu.VMEM((_KG, _G, D), jnp.float32)]
        + [pltpu.SemaphoreType.DMA for _ in range(B + 2 * _NR + 1)]
    )

    @functools.partial(
        pl.kernel,
        out_type=jax.ShapeDtypeStruct((B, S, D), jnp.float32),
        mesh=mesh,
        scratch_types=scratch,
    )
    def emb(ids_hbm, wt_hbm, pt_hbm, out_hbm, *sc):
        idx_bufs = sc[0:B]
        row_bufs = [sc[B + s * B:B + (s + 1) * B] for s in range(_NR)]
        pos_buf = sc[B + _NR * B]
        lsems = sc[B + _NR * B + 1:2 * B + _NR * B + 1]
        gsems = sc[2 * B + _NR * B + 1:2 * B + _NR * B + 1 + _NR]
        osems = sc[2 * B + _NR * B + 1 + _NR:2 * B + _NR * B + 1 + 2 * _NR]
        psem = sc[2 * B + _NR * B + 1 + 2 * _NR]

        wid = lax.axis_index("s") * _NC + lax.axis_index("c")
        row0 = wid * R

        idxd = [
            pltpu.async_copy(ids_hbm.at[b, pl.ds(row0, R)], idx_bufs[b],
                             lsems[b])
            for b in range(B)
        ]
        posd = pltpu.async_copy(pt_hbm.at[pl.ds(wid * _KG, _KG)], pos_buf,
                                psem)

        def start_gathers(k):
            st = k % _NR
            return [
                pltpu.async_copy(
                    wt_hbm.at[idx_bufs[b].at[pl.ds(k * _G, _G)]],
                    row_bufs[st][b], gsems[st])
                for b in range(B)
            ]

        def add_pos(k):
            st = k % _NR

            @plsc.parallel_loop(0, _G, unroll=4)
            def body(r):
                for c0 in range(0, D, _L):
                    v = pos_buf[k, r, pl.ds(c0, _L)]
                    for b in range(B):
                        plsc.addupdate(row_bufs[st][b].at[r, pl.ds(c0, _L)],
                                       v)

        for d in idxd:
            d.wait()
        gath, stores = {}, {}
        gath[0] = start_gathers(0)
        gath[1] = start_gathers(1)
        posd.wait()

        for k in range(_KG):
            st = k % _NR
            g = k + 2
            if g < _KG:
                if g >= _NR:
                    for d in stores[g - _NR]:
                        d.wait()
                gath[g] = start_gathers(g)
            for d in gath[k]:
                d.wait()
            add_pos(k)
            stores[k] = [
                pltpu.async_copy(row_bufs[st][b],
                                 out_hbm.at[b, pl.ds(row0 + k * _G, _G)],
                                 osems[st])
                for b in range(B)
            ]

        for k in range(_KG - _NR, _KG):
            for d in stores[k]:
                d.wait()

    return emb(input_ids, word_table, pos_r)


# speedup vs baseline: 1.0095x; 1.0095x over previous
import functools

import jax
import jax.numpy as jnp
from jax import lax
from jax.experimental import pallas as pl
from jax.experimental.pallas import tpu as pltpu
from jax.experimental.pallas import tpu_sc as plsc

_NC, _NS = 2, 16
_NW = _NC * _NS
_G = 128
_CB = 2
_L = 16


def kernel(input_ids, word_table, pos_table):
    B, S = input_ids.shape
    V, D = word_table.shape
    N = B * S
    NBLK = N // _G
    PBLK = S // _G
    R = _CB * _G
    assert _NW * R == S
    niter = B

    pos_r = pos_table.reshape(PBLK, _G, D)

    mesh = plsc.VectorSubcoreMesh(core_axis_name="c", subcore_axis_name="s")

    scratch = (
        [pltpu.VMEM((R,), jnp.int32) for _ in range(niter)]
        + [pltpu.VMEM((_CB, _G, D), jnp.float32) for _ in range(2)]
        + [pltpu.VMEM((_CB, _G, D), jnp.float32)]
        + [pltpu.SemaphoreType.DMA for _ in range(niter + 2 + 2 + 1)]
    )

    @functools.partial(
        pl.kernel,
        out_type=jax.ShapeDtypeStruct((B, S, D), jnp.float32),
        mesh=mesh,
        scratch_types=scratch,
    )
    def emb(ids_hbm, wt_hbm, pt_hbm, out_hbm, *sc):
        idx_bufs = sc[0:niter]
        row_bufs = sc[niter:niter + 2]
        pos_buf = sc[niter + 2]
        lsems = sc[niter + 3:2 * niter + 3]
        gsems = sc[2 * niter + 3:2 * niter + 5]
        osems = sc[2 * niter + 5:2 * niter + 7]
        psem = sc[2 * niter + 7]

        wid = lax.axis_index("s") * _NC + lax.axis_index("c")
        pblk0 = wid * _CB

        idxd = [
            pltpu.async_copy(ids_hbm.at[c, pl.ds(pblk0 * _G, R)],
                             idx_bufs[c], lsems[c])
            for c in range(niter)
        ]
        posd = pltpu.async_copy(pt_hbm.at[pl.ds(pblk0, _CB)], pos_buf, psem)

        def start_gathers(i):
            b = i % 2
            return [
                pltpu.async_copy(wt_hbm.at[idx_bufs[i].at[pl.ds(k * _G, _G)]],
                                 row_bufs[b].at[k], gsems[b])
                for k in range(_CB)
            ]

        def add_pos_block(i, blkk):
            b = i % 2

            @plsc.parallel_loop(0, _G, unroll=8)
            def body(r):
                for c0 in range(0, D, _L):
                    v = pos_buf[blkk, r, pl.ds(c0, _L)]
                    plsc.addupdate(
                        row_bufs[b].at[blkk, r, pl.ds(c0, _L)], v)

        gath, stores = {}, {}
        idxd[0].wait()
        gath[0] = start_gathers(0)

        for i in range(niter):
            b = i % 2
            if i + 1 < niter:
                idxd[i + 1].wait()
                if i >= 1:
                    for d in stores[i - 1]:
                        d.wait()
                gath[i + 1] = start_gathers(i + 1)
            if i == 0:
                posd.wait()
            blk_stores = []
            for k in range(_CB):
                gath[i][k].wait()
                add_pos_block(i, k)
                blk_stores.append(pltpu.async_copy(
                    row_bufs[b].at[k],
                    out_hbm.at[i, pl.ds((pblk0 + k) * _G, _G)], osems[b]))
            stores[i] = blk_stores

        for i in range(max(0, niter - 2), niter):
            for d in stores[i]:
                d.wait()

    return emb(input_ids, word_table, pos_r)


# speedup vs baseline: 1.0779x; 1.0678x over previous
import functools

import jax
import jax.numpy as jnp
from jax import lax
from jax.experimental import pallas as pl
from jax.experimental.pallas import tpu as pltpu
from jax.experimental.pallas import tpu_sc as plsc

_NC, _NS = 2, 16
_NW = _NC * _NS
_G = 128
_CB = 2
_L = 16


def kernel(input_ids, word_table, pos_table):
    B, S = input_ids.shape
    V, D = word_table.shape
    N = B * S
    NBLK = N // _G
    PBLK = S // _G
    R = _CB * _G
    assert _NW * R == S
    niter = B

    pos_r = pos_table.reshape(PBLK, _G, D)

    mesh = plsc.VectorSubcoreMesh(core_axis_name="c", subcore_axis_name="s")

    scratch = (
        [pltpu.VMEM((R,), jnp.int32) for _ in range(niter)]
        + [pltpu.VMEM((_CB, _G, D), jnp.float32) for _ in range(2)]
        + [pltpu.VMEM((_CB, _G, D), jnp.float32)]
        + [pltpu.SemaphoreType.DMA for _ in range(niter + 2 + 2 + 1)]
    )

    @functools.partial(
        pl.kernel,
        out_type=jax.ShapeDtypeStruct((B, S, D), jnp.float32),
        mesh=mesh,
        scratch_types=scratch,
    )
    def emb(ids_hbm, wt_hbm, pt_hbm, out_hbm, *sc):
        idx_bufs = sc[0:niter]
        row_bufs = sc[niter:niter + 2]
        pos_buf = sc[niter + 2]
        lsems = sc[niter + 3:2 * niter + 3]
        gsems = sc[2 * niter + 3:2 * niter + 5]
        osems = sc[2 * niter + 5:2 * niter + 7]
        psem = sc[2 * niter + 7]

        wid = lax.axis_index("s") * _NC + lax.axis_index("c")
        pblk0 = wid * _CB

        idxd = [
            pltpu.async_copy(ids_hbm.at[c, pl.ds(pblk0 * _G, R)],
                             idx_bufs[c], lsems[c])
            for c in range(niter)
        ]
        posd = pltpu.async_copy(pt_hbm.at[pl.ds(pblk0, _CB)], pos_buf, psem)

        def start_gathers(i):
            b = i % 2
            return [
                pltpu.async_copy(wt_hbm.at[idx_bufs[i].at[pl.ds(k * _G, _G)]],
                                 row_bufs[b].at[k], gsems[b])
                for k in range(_CB)
            ]

        def add_pos_block(i, blkk):
            b = i % 2

            @plsc.parallel_loop(0, _G, unroll=4)
            def body(r):
                for c0 in range(0, D, _L):
                    v = pos_buf[blkk, r, pl.ds(c0, _L)]
                    plsc.addupdate(
                        row_bufs[b].at[blkk, r, pl.ds(c0, _L)], v)

        gath, stores = {}, {}
        idxd[0].wait()
        gath[0] = start_gathers(0)

        for i in range(niter):
            b = i % 2
            if i + 1 < niter:
                idxd[i + 1].wait()
                if i >= 1:
                    for d in stores[i - 1]:
                        d.wait()
                gath[i + 1] = start_gathers(i + 1)
            if i == 0:
                posd.wait()
            blk_stores = []
            for k in range(_CB):
                gath[i][k].wait()
                add_pos_block(i, k)
                blk_stores.append(pltpu.async_copy(
                    row_bufs[b].at[k],
                    out_hbm.at[i, pl.ds((pblk0 + k) * _G, _G)], osems[b]))
            stores[i] = blk_stores

        for i in range(max(0, niter - 2), niter):
            for d in stores[i]:
                d.wait()

    return emb(input_ids, word_table, pos_r)
